# baseline (speedup 1.0000x reference)
import functools

import jax
import jax.numpy as jnp
from jax import lax
from jax.experimental import pallas as pl
from jax.experimental.pallas import tpu as pltpu
from jax.experimental.pallas import tpu_sc as plsc

_V = 1000000
_D = 64
_B = 16384
_L = 200
_NI = 4
_ID = 64
_H = 128

_NC = 2
_NS = 16
_NW = _NC * _NS
_G = 64


def _make_pool(nb):
  rw = nb // _NW
  ng = rw // _G

  @functools.partial(
    pl.kernel,
    out_type=jax.ShapeDtypeStruct((nb, _D), jnp.float32),
    mesh=plsc.VectorSubcoreMesh(core_axis_name="c", subcore_axis_name="s"),
    scratch_types=[
        pltpu.VMEM((2, _G, _L), jnp.int32),
        pltpu.VMEM((4, _L, _D), jnp.float32),
        pltpu.VMEM((_G, _D), jnp.float32),
        pltpu.SemaphoreType.DMA,
        pltpu.SemaphoreType.DMA,
        pltpu.SemaphoreType.DMA,
        pltpu.SemaphoreType.DMA,
        pltpu.SemaphoreType.DMA,
    ],
    compiler_params=pltpu.CompilerParams(use_tc_tiling_on_sc=False),
  )
  def _sc_pool(emb_hbm, text_hbm, out_hbm, idx_v, rows_v, pooled_v,
               sem0, sem1, sem2, sem3, idx_sem):
    _NG = ng
    wid = lax.axis_index("s") * _NC + lax.axis_index("c")
    base = wid * rw
    scale = jnp.float32(1.0 / _L)
    sems = (sem0, sem1, sem2, sem3)

    def fire(q, r, slot):
        pltpu.async_copy(
            emb_hbm.at[idx_v.at[q, r]],
            rows_v.at[slot], sems[slot])

    def wait_rows(q, slot):
        pltpu.make_async_copy(
            emb_hbm.at[idx_v.at[q, 0]],
            rows_v.at[slot], sems[slot]).wait()

    def accum_to(slot, r):
        def accum(j, acc):
            return (
                acc[0] + rows_v[slot, j, pl.ds(0, 16)],
                acc[1] + rows_v[slot, j, pl.ds(16, 16)],
                acc[2] + rows_v[slot, j, pl.ds(32, 16)],
                acc[3] + rows_v[slot, j, pl.ds(48, 16)],
                acc[4] + rows_v[slot, j + 1, pl.ds(0, 16)],
                acc[5] + rows_v[slot, j + 1, pl.ds(16, 16)],
                acc[6] + rows_v[slot, j + 1, pl.ds(32, 16)],
                acc[7] + rows_v[slot, j + 1, pl.ds(48, 16)],
            )
        z = jnp.zeros((16,), jnp.float32)
        a8 = plsc.parallel_loop(0, _L, step=2, unroll=4, carry=(z,) * 8)(accum)
        a = (a8[0] + a8[4], a8[1] + a8[5], a8[2] + a8[6], a8[3] + a8[7])
        pooled_v[r, pl.ds(0, 16)] = a[0] * scale
        pooled_v[r, pl.ds(16, 16)] = a[1] * scale
        pooled_v[r, pl.ds(32, 16)] = a[2] * scale
        pooled_v[r, pl.ds(48, 16)] = a[3] * scale

    pltpu.sync_copy(text_hbm.at[pl.ds(base, _G)], idx_v.at[0])

    for g in range(_NG):
        q = g & 1
        row0 = base + g * _G
        if g + 1 < _NG:
            pltpu.async_copy(
                text_hbm.at[pl.ds(row0 + _G, _G)],
                idx_v.at[1 - q], idx_sem)
        fire(q, 0, 0)
        fire(q, 1, 1)
        fire(q, 2, 2)

        def quad(rr, carry):
            r = pl.multiple_of(rr * 4, 4)
            for k in range(4):
                fire(q, r + k + 3, (k + 3) & 3)
                wait_rows(q, k)
                accum_to(k, r + k)
            return carry

        lax.fori_loop(0, _G // 4 - 1, quad, 0)
        r = _G - 4
        fire(q, _G - 1, 3)
        for k in range(4):
            wait_rows(q, k)
            accum_to(k, r + k)

        pltpu.sync_copy(pooled_v, out_hbm.at[pl.ds(row0, _G)])
        if g + 1 < _NG:
            pltpu.make_async_copy(
                text_hbm.at[pl.ds(base, _G)],
                idx_v.at[1 - q], idx_sem).wait()

  return _sc_pool


_pool_quarter = _make_pool(_B // 4)

_BB = 4096


def _mlp_body(p_ref, w0_ref, b0_ref, w1_ref, b1_ref, w2_ref, b2_ref,
              w3_ref, b3_ref, wf_ref, bf_ref, wo_ref, bo_ref, out_ref):
    p = p_ref[...]
    h0 = jnp.maximum(jnp.dot(p, w0_ref[...],
                             preferred_element_type=jnp.float32) + b0_ref[...], 0.0)
    h1 = jnp.maximum(jnp.dot(p, w1_ref[...],
                             preferred_element_type=jnp.float32) + b1_ref[...], 0.0)
    h2 = jnp.maximum(jnp.dot(p, w2_ref[...],
                             preferred_element_type=jnp.float32) + b2_ref[...], 0.0)
    h3 = jnp.maximum(jnp.dot(p, w3_ref[...],
                             preferred_element_type=jnp.float32) + b3_ref[...], 0.0)
    f = (jnp.dot(h0, wf_ref[0:_ID, :], preferred_element_type=jnp.float32)
         + jnp.dot(h1, wf_ref[_ID:2 * _ID, :], preferred_element_type=jnp.float32)
         + jnp.dot(h2, wf_ref[2 * _ID:3 * _ID, :], preferred_element_type=jnp.float32)
         + jnp.dot(h3, wf_ref[3 * _ID:4 * _ID, :], preferred_element_type=jnp.float32)
         + bf_ref[...])
    fused = jnp.maximum(f, 0.0)
    out_ref[...] = (jnp.sum(fused * wo_ref[...], axis=1, keepdims=True)
                    + bo_ref[...])


def _tc_mlp(pooled, W0, b0, W1, b1, W2, b2, W3, b3, Wf, bf, wo_row, bo):
    nb = pooled.shape[0]
    full = lambda shape: pl.BlockSpec(shape, lambda i: (0, 0))
    return pl.pallas_call(
        _mlp_body,
        grid=(nb // _BB,),
        in_specs=[
            pl.BlockSpec((_BB, _D), lambda i: (i, 0)),
            full((_D, _ID)), full((1, _ID)),
            full((_D, _ID)), full((1, _ID)),
            full((_D, _ID)), full((1, _ID)),
            full((_D, _ID)), full((1, _ID)),
            full((_NI * _ID, _H)), full((1, _H)),
            full((1, _H)), full((1, 1)),
        ],
        out_specs=pl.BlockSpec((_BB, 1), lambda i: (i, 0)),
        out_shape=jax.ShapeDtypeStruct((nb, 1), jnp.float32),
    )(pooled, W0, b0, W1, b1, W2, b2, W3, b3, Wf, bf, wo_row, bo)


def kernel(text, emb, W0, b0, W1, b1, W2, b2, W3, b3, Wf, bf, Wo, bo):
    qb = _B // 4
    mlp_args = (W0, b0.reshape(1, -1), W1, b1.reshape(1, -1),
                W2, b2.reshape(1, -1), W3, b3.reshape(1, -1),
                Wf, bf.reshape(1, -1), Wo.reshape(1, _H), bo.reshape(1, 1))
    outs = [_tc_mlp(_pool_quarter(emb, text[k * qb:(k + 1) * qb]), *mlp_args)
            for k in range(4)]
    return jnp.concatenate(outs, axis=0)
